# baseline (speedup 1.0000x reference)
import functools

import jax
import jax.numpy as jnp
from jax import lax
from jax.experimental import pallas as pl
from jax.experimental.pallas import tpu as pltpu
from jax.experimental.pallas import tpu_sc as plsc

B = 16384
STYLE_DIM = 64
D = 2 * STYLE_DIM
OUT_DIM = 128
N_COMBO_PAD = 16

_INV_SQRT2 = 0.7071067811865476


def _mlp_table_body(
    gt_ref, at_ref, w1_ref, b1_ref, w2_ref, b2_ref, g2d_ref, a2d_ref,
    out_ref, idx_ref
):
    idx_ref[...] = g2d_ref[...] * 3 + a2d_ref[...]
    r = lax.broadcasted_iota(jnp.int32, (N_COMBO_PAD, 3), 0)
    j = lax.broadcasted_iota(jnp.int32, (N_COMBO_PAD, 3), 1)
    onehot_g = (r // 3 == j).astype(jnp.float32)
    onehot_a = (r % 3 == j).astype(jnp.float32)
    left = jnp.dot(onehot_g, gt_ref[...], preferred_element_type=jnp.float32)
    right = jnp.dot(onehot_a, at_ref[...], preferred_element_type=jnp.float32)
    z = (
        jnp.dot(left, w1_ref[0:STYLE_DIM, :], preferred_element_type=jnp.float32)
        + jnp.dot(right, w1_ref[STYLE_DIM:D, :], preferred_element_type=jnp.float32)
        + b1_ref[...]
    )
    g = 0.5 * z * (1.0 + lax.erf(z * _INV_SQRT2))
    out_ref[...] = (
        jnp.dot(g, w2_ref[...], preferred_element_type=jnp.float32) + b2_ref[...]
    )


_mlp_table = pl.pallas_call(
    _mlp_table_body,
    out_shape=(
        jax.ShapeDtypeStruct((N_COMBO_PAD, OUT_DIM), jnp.float32),
        jax.ShapeDtypeStruct((B // 128, 128), jnp.int32),
    ),
)

_NC = 2
_NS = 16
_L = 16
_NW = _NC * _NS
_BPW = B // _NW
_CHUNK = 128
_NCHUNK = _BPW // _CHUNK


def _sc_gather_body(table_hbm, idx_hbm, out_hbm, iv, rows, shared_tab, gsem, osem):
    sid = lax.axis_index("s")
    wid = sid * _NC + lax.axis_index("c")
    base = wid * _BPW
    idx_cp = pltpu.async_copy(
        idx_hbm.at[pl.ds(wid * _NCHUNK, _NCHUNK)], iv, osem
    )
    @pl.when(sid == 0)
    def _load_table():
        pltpu.sync_copy(table_hbm, shared_tab)

    idx_cp.wait()
    plsc.subcore_barrier()
    gather_cps = [
        pltpu.async_copy(
            shared_tab.at[iv.at[c]],
            rows.at[pl.ds(c * _CHUNK, _CHUNK)],
            gsem.at[c],
        )
        for c in range(_NCHUNK)
    ]
    out_cps = []
    for c in range(_NCHUNK):
        gather_cps[c].wait()
        out_cps.append(
            pltpu.async_copy(
                rows.at[pl.ds(c * _CHUNK, _CHUNK)],
                out_hbm.at[pl.ds(base + c * _CHUNK, _CHUNK)],
                osem,
            )
        )
    for cp in out_cps:
        cp.wait()


@functools.cache
def _sc_gather():
    mesh = plsc.VectorSubcoreMesh(
        core_axis_name="c", subcore_axis_name="s", num_cores=_NC, num_subcores=_NS
    )
    return pl.kernel(
        _sc_gather_body,
        mesh=mesh,
        out_type=jax.ShapeDtypeStruct((B, OUT_DIM), jnp.float32),
        name="combo_gather",
        scratch_types=[
            pltpu.VMEM((_NCHUNK, _CHUNK), jnp.int32),
            pltpu.VMEM((_BPW, OUT_DIM), jnp.float32),
            pltpu.VMEM_SHARED((N_COMBO_PAD, OUT_DIM), jnp.float32),
            pltpu.SemaphoreType.DMA((_NCHUNK,)),
            pltpu.SemaphoreType.DMA,
        ],
    )


def kernel(gender_ids, age_ids, gender_table, age_table, W1, b1, W2, b2):
    g2d = gender_ids.astype(jnp.int32).reshape(B // 128, 128)
    a2d = age_ids.astype(jnp.int32).reshape(B // 128, 128)
    table, idx = _mlp_table(
        gender_table, age_table, W1, b1.reshape(1, D), W2, b2.reshape(1, OUT_DIM),
        g2d, a2d,
    )
    return _sc_gather()(table, idx)


# speedup vs baseline: 1.0120x; 1.0120x over previous
import functools

import jax
import jax.numpy as jnp
from jax import lax
from jax.experimental import pallas as pl
from jax.experimental.pallas import tpu as pltpu
from jax.experimental.pallas import tpu_sc as plsc

B = 16384
STYLE_DIM = 64
D = 2 * STYLE_DIM
OUT_DIM = 128
N_COMBO_PAD = 16

_INV_SQRT2 = 0.7071067811865476


def _mlp_table_body(gt_ref, at_ref, w1_ref, b1_ref, w2_ref, b2_ref, out_ref):
    r = lax.broadcasted_iota(jnp.int32, (N_COMBO_PAD, 3), 0)
    j = lax.broadcasted_iota(jnp.int32, (N_COMBO_PAD, 3), 1)
    onehot_g = (r // 3 == j).astype(jnp.float32)
    onehot_a = (r % 3 == j).astype(jnp.float32)
    left = jnp.dot(onehot_g, gt_ref[...], preferred_element_type=jnp.float32)
    right = jnp.dot(onehot_a, at_ref[...], preferred_element_type=jnp.float32)
    z = (
        jnp.dot(left, w1_ref[0:STYLE_DIM, :], preferred_element_type=jnp.float32)
        + jnp.dot(right, w1_ref[STYLE_DIM:D, :], preferred_element_type=jnp.float32)
        + b1_ref[...]
    )
    g = 0.5 * z * (1.0 + lax.erf(z * _INV_SQRT2))
    out_ref[...] = (
        jnp.dot(g, w2_ref[...], preferred_element_type=jnp.float32) + b2_ref[...]
    )


_mlp_table = pl.pallas_call(
    _mlp_table_body,
    out_shape=jax.ShapeDtypeStruct((N_COMBO_PAD, OUT_DIM), jnp.float32),
)

_NC = 2
_NS = 16
_L = 16
_NW = _NC * _NS
_BPW = B // _NW
_CHUNK = 64
_NCHUNK = _BPW // _CHUNK


def _sc_gather_body(
    table_hbm, g_hbm, a_hbm, out_hbm, gv, av, iv, rows, shared_tab, gsem, osem
):
    sid = lax.axis_index("s")
    wid = sid * _NC + lax.axis_index("c")
    base = wid * _BPW
    id_cps = [
        pltpu.async_copy(g_hbm.at[pl.ds(base, _BPW)], gv, osem),
        pltpu.async_copy(a_hbm.at[pl.ds(base, _BPW)], av, osem),
    ]
    @pl.when(sid == 0)
    def _load_table():
        pltpu.sync_copy(table_hbm, shared_tab)

    for cp in id_cps:
        cp.wait()
    for c in range(_NCHUNK):
        for i in range(_CHUNK // _L):
            s = c * _CHUNK + i * _L
            iv[c, pl.ds(i * _L, _L)] = gv[pl.ds(s, _L)] * 3 + av[pl.ds(s, _L)]
    plsc.subcore_barrier()
    gather_cps = [
        pltpu.async_copy(
            shared_tab.at[iv.at[c]],
            rows.at[pl.ds(c * _CHUNK, _CHUNK)],
            gsem.at[c],
        )
        for c in range(_NCHUNK)
    ]
    out_cps = []
    for c in range(_NCHUNK):
        gather_cps[c].wait()
        out_cps.append(
            pltpu.async_copy(
                rows.at[pl.ds(c * _CHUNK, _CHUNK)],
                out_hbm.at[pl.ds(base + c * _CHUNK, _CHUNK)],
                osem,
            )
        )
    for cp in out_cps:
        cp.wait()


@functools.cache
def _sc_gather():
    mesh = plsc.VectorSubcoreMesh(
        core_axis_name="c", subcore_axis_name="s", num_cores=_NC, num_subcores=_NS
    )
    return pl.kernel(
        _sc_gather_body,
        mesh=mesh,
        out_type=jax.ShapeDtypeStruct((B, OUT_DIM), jnp.float32),
        name="combo_gather",
        scratch_types=[
            pltpu.VMEM((_BPW,), jnp.int32),
            pltpu.VMEM((_BPW,), jnp.int32),
            pltpu.VMEM((_NCHUNK, _CHUNK), jnp.int32),
            pltpu.VMEM((_BPW, OUT_DIM), jnp.float32),
            pltpu.VMEM_SHARED((N_COMBO_PAD, OUT_DIM), jnp.float32),
            pltpu.SemaphoreType.DMA((_NCHUNK,)),
            pltpu.SemaphoreType.DMA,
        ],
    )


def kernel(gender_ids, age_ids, gender_table, age_table, W1, b1, W2, b2):
    table = _mlp_table(
        gender_table, age_table, W1, b1.reshape(1, D), W2, b2.reshape(1, OUT_DIM)
    )
    return _sc_gather()(
        table, gender_ids.astype(jnp.int32), age_ids.astype(jnp.int32)
    )
